# baseline (speedup 1.0000x reference)
import jax
import jax.numpy as jnp
from jax import lax
from jax.experimental import pallas as pl
from jax.experimental.pallas import tpu as pltpu
from jax.experimental.pallas import tpu_sc as plsc

N_ENT = 100000
N_REL = 500
D = 128
BATCH = 16384
N_DAY = 368

NC = 2
NS = 16
NW = NC * NS
ROWS_PER_W = BATCH // NW
CHUNK = 128
NCHUNK = ROWS_PER_W // CHUNK
NBUF = 7
NGATHER = 5


def _sc_gather_body(heads_hbm, tails_hbm, e1_hbm, e2_hbm,
                    o_hr, o_tr, o_hi, o_ti,
                    idx_h, idx_t, bufs, gsems, ssems):
    c = lax.axis_index("c")
    s = lax.axis_index("s")
    wid = s * NC + c
    base = wid * ROWS_PER_W

    pltpu.sync_copy(heads_hbm.at[wid], idx_h)
    pltpu.sync_copy(tails_hbm.at[wid], idx_t)

    seq = [(e1_hbm, idx_h, o_hr), (e1_hbm, idx_t, o_tr),
           (e2_hbm, idx_h, o_hi), (e2_hbm, idx_t, o_ti)]
    transfers = [(tbl, idx, out, j) for (tbl, idx, out) in seq
                 for j in range(NCHUNK)]
    n = len(transfers)
    gh = [None] * NBUF
    sh = [None] * NBUF

    def start_gather(g):
        tbl, idx, out, j = transfers[g]
        b = g % NBUF
        gh[b] = pltpu.async_copy(tbl.at[idx.at[j]], bufs[b], gsems[b])

    for g in range(NGATHER):
        start_gather(g)
    for k in range(n):
        b = k % NBUF
        gh[b].wait()
        tbl, idx, out, j = transfers[k]
        sh[b] = pltpu.async_copy(bufs[b], out.at[pl.ds(base + j * CHUNK, CHUNK)],
                                 ssems[b])
        g = k + NGATHER
        if g < n:
            bg = g % NBUF
            if sh[bg] is not None:
                sh[bg].wait()
                sh[bg] = None
            start_gather(g)
    for b in range(NBUF):
        if sh[b] is not None:
            sh[b].wait()


def _sc_gather(heads_r, tails_r, e1, e2):
    out = jax.ShapeDtypeStruct((BATCH, D), jnp.float32)
    return pl.kernel(
        _sc_gather_body,
        out_type=[out, out, out, out],
        mesh=plsc.VectorSubcoreMesh(core_axis_name="c", subcore_axis_name="s"),
        scratch_types=[
            pltpu.VMEM((NCHUNK, CHUNK), jnp.int32),
            pltpu.VMEM((NCHUNK, CHUNK), jnp.int32),
            [pltpu.VMEM((CHUNK, D), jnp.float32) for _ in range(NBUF)],
            [pltpu.SemaphoreType.DMA for _ in range(NBUF)],
            [pltpu.SemaphoreType.DMA for _ in range(NBUF)],
        ],
    )(heads_r, tails_r, e1, e2)


def _trig_body(w_ref, t_ref):
    dayf = lax.broadcasted_iota(jnp.int32, (N_DAY, D), 0).astype(jnp.float32)
    phase = dayf * w_ref[...]
    t_ref[:, :D] = jnp.cos(phase).astype(jnp.bfloat16)
    t_ref[:, D:] = jnp.sin(phase).astype(jnp.bfloat16)


def _trig_tables(w2):
    out = jax.ShapeDtypeStruct((N_DAY, 2 * D), jnp.bfloat16)
    return pl.pallas_call(_trig_body, out_shape=out)(w2)


BLK = 1024
NB = BATCH // BLK
_DIMS0 = (((0,), (0,)), ((), ()))


def _tc_rel_body(rels_ref, day_ref, relcat_ref, trig_ref, or_ref, oi_ref):
    rels = rels_ref[0]
    day = day_ref[0]
    oh_rel = (rels == lax.broadcasted_iota(jnp.int32, (N_REL, BLK), 0)
              ).astype(jnp.bfloat16)
    oh_day = (day == lax.broadcasted_iota(jnp.int32, (N_DAY, BLK), 0)
              ).astype(jnp.bfloat16)
    rr = lax.dot_general(oh_rel, relcat_ref[...], _DIMS0,
                         preferred_element_type=jnp.float32)
    tt = lax.dot_general(oh_day, trig_ref[...], _DIMS0,
                         preferred_element_type=jnp.float32)
    r1 = rr[:, :D]
    r2 = rr[:, D:]
    d_real = tt[:, :D]
    d_img = tt[:, D:]
    or_ref[...] = d_real * r1 - d_img * r2
    oi_ref[...] = d_real * r2 + d_img * r1


def _tc_rel(rels3, day3, relcat, trig):
    out = jax.ShapeDtypeStruct((BATCH, D), jnp.float32)
    return pl.pallas_call(
        _tc_rel_body,
        grid=(NB,),
        in_specs=[
            pl.BlockSpec((1, 1, BLK), lambda i: (i, 0, 0)),
            pl.BlockSpec((1, 1, BLK), lambda i: (i, 0, 0)),
            pl.BlockSpec((N_REL, 2 * D), lambda i: (0, 0)),
            pl.BlockSpec((N_DAY, 2 * D), lambda i: (0, 0)),
        ],
        out_specs=[
            pl.BlockSpec((BLK, D), lambda i: (i, 0)),
            pl.BlockSpec((BLK, D), lambda i: (i, 0)),
        ],
        out_shape=[out, out],
    )(rels3, day3, relcat, trig)


def kernel(heads, rels, tails, day, ent_emb1, ent_emb2, rel_emb1, rel_emb2, w):
    heads_r = heads.astype(jnp.int32).reshape(NW, NCHUNK, CHUNK)
    tails_r = tails.astype(jnp.int32).reshape(NW, NCHUNK, CHUNK)
    rels3 = rels.astype(jnp.int32).reshape(NB, 1, BLK)
    day3 = day.astype(jnp.int32).reshape(NB, 1, BLK)
    w2 = w.reshape(1, D)

    h_r, t_r, h_i, t_i = _sc_gather(heads_r, tails_r, ent_emb1, ent_emb2)
    trig = _trig_tables(w2)
    relcat = jnp.concatenate([rel_emb1, rel_emb2], axis=1).astype(jnp.bfloat16)
    r_r, r_i = _tc_rel(rels3, day3, relcat, trig)
    return (h_r, r_r, t_r, h_i, r_i, t_i)


# speedup vs baseline: 1.0139x; 1.0139x over previous
import jax
import jax.numpy as jnp
from jax import lax
from jax.experimental import pallas as pl
from jax.experimental.pallas import tpu as pltpu
from jax.experimental.pallas import tpu_sc as plsc

N_ENT = 100000
N_REL = 500
D = 128
BATCH = 16384
N_DAY = 368

NC = 2
NS = 16
NW = NC * NS
ROWS_PER_W = BATCH // NW
CHUNK = 128
NCHUNK = ROWS_PER_W // CHUNK
NBUF = 7
NGATHER = 3


def _sc_gather_body(ht_hbm, e1_hbm, e2_hbm,
                    o_hr, o_tr, o_hi, o_ti,
                    idx_ht, bufs, gsems, ssems):
    c = lax.axis_index("c")
    s = lax.axis_index("s")
    wid = s * NC + c
    base = wid * ROWS_PER_W

    pltpu.sync_copy(ht_hbm.at[wid], idx_ht)

    seq = [(e1_hbm, 0, o_hr), (e1_hbm, NCHUNK, o_tr),
           (e2_hbm, 0, o_hi), (e2_hbm, NCHUNK, o_ti)]
    transfers = [(tbl, off + j, out, j) for (tbl, off, out) in seq
                 for j in range(NCHUNK)]
    n = len(transfers)
    gh = [None] * NBUF
    sh = [None] * NBUF

    def start_gather(g):
        tbl, row, out, j = transfers[g]
        b = g % NBUF
        gh[b] = pltpu.async_copy(tbl.at[idx_ht.at[row]], bufs[b], gsems[b])

    for g in range(NGATHER):
        start_gather(g)
    for k in range(n):
        b = k % NBUF
        gh[b].wait()
        tbl, row, out, j = transfers[k]
        sh[b] = pltpu.async_copy(bufs[b], out.at[pl.ds(base + j * CHUNK, CHUNK)],
                                 ssems[b])
        g = k + NGATHER
        if g < n:
            bg = g % NBUF
            if sh[bg] is not None:
                sh[bg].wait()
                sh[bg] = None
            start_gather(g)
    for b in range(NBUF):
        if sh[b] is not None:
            sh[b].wait()


def _sc_gather(ht_r, e1, e2):
    out = jax.ShapeDtypeStruct((BATCH, D), jnp.float32)
    return pl.kernel(
        _sc_gather_body,
        out_type=[out, out, out, out],
        mesh=plsc.VectorSubcoreMesh(core_axis_name="c", subcore_axis_name="s"),
        scratch_types=[
            pltpu.VMEM((2 * NCHUNK, CHUNK), jnp.int32),
            [pltpu.VMEM((CHUNK, D), jnp.float32) for _ in range(NBUF)],
            [pltpu.SemaphoreType.DMA for _ in range(NBUF)],
            [pltpu.SemaphoreType.DMA for _ in range(NBUF)],
        ],
    )(ht_r, e1, e2)


def _trig_body(w_ref, t_ref):
    dayf = lax.broadcasted_iota(jnp.int32, (N_DAY, D), 0).astype(jnp.float32)
    phase = dayf * w_ref[...]
    t_ref[:, :D] = jnp.cos(phase).astype(jnp.bfloat16)
    t_ref[:, D:] = jnp.sin(phase).astype(jnp.bfloat16)


def _trig_tables(w2):
    out = jax.ShapeDtypeStruct((N_DAY, 2 * D), jnp.bfloat16)
    return pl.pallas_call(_trig_body, out_shape=out)(w2)


BLK = 1024
NB = BATCH // BLK
_DIMS0 = (((0,), (0,)), ((), ()))


def _tc_rel_body(rels_ref, day_ref, relcat_ref, trig_ref, or_ref, oi_ref):
    rels = rels_ref[0]
    day = day_ref[0]
    oh_rel = (rels == lax.broadcasted_iota(jnp.int32, (N_REL, BLK), 0)
              ).astype(jnp.bfloat16)
    oh_day = (day == lax.broadcasted_iota(jnp.int32, (N_DAY, BLK), 0)
              ).astype(jnp.bfloat16)
    rr = lax.dot_general(oh_rel, relcat_ref[...], _DIMS0,
                         preferred_element_type=jnp.float32)
    tt = lax.dot_general(oh_day, trig_ref[...], _DIMS0,
                         preferred_element_type=jnp.float32)
    r1 = rr[:, :D]
    r2 = rr[:, D:]
    d_real = tt[:, :D]
    d_img = tt[:, D:]
    or_ref[...] = d_real * r1 - d_img * r2
    oi_ref[...] = d_real * r2 + d_img * r1


def _tc_rel(rels3, day3, relcat, trig):
    out = jax.ShapeDtypeStruct((BATCH, D), jnp.float32)
    return pl.pallas_call(
        _tc_rel_body,
        grid=(NB,),
        in_specs=[
            pl.BlockSpec((1, 1, BLK), lambda i: (i, 0, 0)),
            pl.BlockSpec((1, 1, BLK), lambda i: (i, 0, 0)),
            pl.BlockSpec((N_REL, 2 * D), lambda i: (0, 0)),
            pl.BlockSpec((N_DAY, 2 * D), lambda i: (0, 0)),
        ],
        out_specs=[
            pl.BlockSpec((BLK, D), lambda i: (i, 0)),
            pl.BlockSpec((BLK, D), lambda i: (i, 0)),
        ],
        out_shape=[out, out],
    )(rels3, day3, relcat, trig)


def kernel(heads, rels, tails, day, ent_emb1, ent_emb2, rel_emb1, rel_emb2, w):
    ht_r = jnp.concatenate(
        [heads.astype(jnp.int32).reshape(NW, NCHUNK, CHUNK),
         tails.astype(jnp.int32).reshape(NW, NCHUNK, CHUNK)], axis=1)
    rels3 = rels.astype(jnp.int32).reshape(NB, 1, BLK)
    day3 = day.astype(jnp.int32).reshape(NB, 1, BLK)
    w2 = w.reshape(1, D)

    h_r, t_r, h_i, t_i = _sc_gather(ht_r, ent_emb1, ent_emb2)
    trig = _trig_tables(w2)
    relcat = jnp.concatenate([rel_emb1, rel_emb2], axis=1).astype(jnp.bfloat16)
    r_r, r_i = _tc_rel(rels3, day3, relcat, trig)
    return (h_r, r_r, t_r, h_i, r_i, t_i)
